# baseline (speedup 1.0000x reference)
import functools

import jax
import jax.numpy as jnp
from jax import lax
from jax.experimental import pallas as pl
from jax.experimental.pallas import tpu as pltpu
from jax.experimental.pallas import tpu_sc as plsc

N = 50000
E = 600000
D = 128
H = 64
NC = 2
NS = 16
LANES = 16
NW = NC * NS

E_PAD = 622592
EPT_SC = E_PAD // NS
EPW = E_PAD // NW
ECHUNK = 2048
NCH_SEG = EPT_SC // ECHUNK
ECH_E = 1024
NCH_EDGE = EPW // ECH_E

ACC = 51200
SENT = 50000
RPT = ACC // NS
ZROWS = 128
NZCOPY = RPT // ZROWS

BN = 400
ACC_BLK = ACC // BN


def _mesh():
    return plsc.VectorSubcoreMesh(
        core_axis_name="c", subcore_axis_name="s",
        num_cores=NC, num_subcores=NS)


_SC_PARAMS = pltpu.CompilerParams(use_tc_tiling_on_sc=False)


def _deg_body(idx_hbm, cnt, idx_v, ones_v, zb_v, acc_sh, semi, sem):
    cid = lax.axis_index("c")
    sid = lax.axis_index("s")

    def fill_ones(i, _):
        ones_v[i, :] = jnp.ones((LANES,), jnp.float32)
        return 0
    lax.fori_loop(0, ECHUNK, fill_ones, 0)

    def fill_zeros(i, _):
        zb_v[i, :] = jnp.zeros((LANES,), jnp.float32)
        return 0
    lax.fori_loop(0, ZROWS, fill_zeros, 0)
    dz = [pltpu.async_copy(
              zb_v, acc_sh.at[pl.ds(sid * RPT + i * ZROWS, ZROWS)], semi)
          for i in range(NZCOPY)]
    for d_ in dz:
        d_.wait()
    plsc.subcore_barrier()

    def chunk(i, _):
        base = cid * E_PAD + sid * EPT_SC + i * ECHUNK
        pltpu.async_copy(idx_hbm.at[pl.ds(base, ECHUNK)], idx_v, semi).wait()
        pltpu.async_copy(ones_v, acc_sh.at[idx_v], sem, add=True).wait()
        return 0
    lax.fori_loop(0, NCH_SEG, chunk, 0)
    plsc.subcore_barrier()

    start = sid * RPT
    pltpu.sync_copy(acc_sh.at[pl.ds(start, RPT)],
                    cnt.at[pl.ds(cid * ACC + start, RPT)])


def _degrees(idx_stacked):
    f = functools.partial(
        pl.kernel,
        out_type=jax.ShapeDtypeStruct((2 * ACC, LANES), jnp.float32),
        mesh=_mesh(),
        compiler_params=_SC_PARAMS,
        scratch_types=[pltpu.VMEM((ECHUNK,), jnp.int32),
                       pltpu.VMEM((ECHUNK, LANES), jnp.float32),
                       pltpu.VMEM((ZROWS, LANES), jnp.float32),
                       pltpu.VMEM_SHARED((ACC, LANES), jnp.float32),
                       pltpu.SemaphoreType.DMA,
                       pltpu.SemaphoreType.DMA],
    )(_deg_body)
    return f(idx_stacked)


def _seg_body(tabm01, tabm23, tabu01, tabu23,
              srcg2, srcs, dstg2, dsts,
              sm01, sm23, su01, su23,
              idxg_v, idxs_v, rows_v, zb_v, acc_sh, semi, sem, sem2):
    cid = lax.axis_index("c")
    sid = lax.axis_index("s")

    def fill_zeros(i, _):
        zb_v[i, :] = jnp.zeros((LANES,), jnp.float32)
        return 0
    lax.fori_loop(0, ZROWS, fill_zeros, 0)

    def one_pass(tab, gidx2, sidx, out):
        dz = [pltpu.async_copy(
                  zb_v, acc_sh.at[pl.ds(sid * RPT + i * ZROWS, ZROWS)], semi)
              for i in range(NZCOPY)]
        for d_ in dz:
            d_.wait()
        plsc.subcore_barrier()

        def chunk(i, _):
            gbase = cid * E_PAD + sid * EPT_SC + i * ECHUNK
            sbase = sid * EPT_SC + i * ECHUNK
            di = [pltpu.async_copy(gidx2.at[pl.ds(gbase, ECHUNK)], idxg_v, semi),
                  pltpu.async_copy(sidx.at[pl.ds(sbase, ECHUNK)], idxs_v, semi)]
            for d_ in di:
                d_.wait()
            pltpu.async_copy(tab.at[idxg_v], rows_v, sem).wait()
            pltpu.async_copy(rows_v, acc_sh.at[idxs_v], sem2, add=True).wait()
            return 0
        lax.fori_loop(0, NCH_SEG, chunk, 0)
        plsc.subcore_barrier()

        start = sid * RPT
        pltpu.sync_copy(acc_sh.at[pl.ds(start, RPT)],
                        out.at[pl.ds(cid * ACC + start, RPT)])
        plsc.subcore_barrier()

    one_pass(tabm01, srcg2, dsts, sm01)
    one_pass(tabm23, srcg2, dsts, sm23)
    one_pass(tabu01, dstg2, srcs, su01)
    one_pass(tabu23, dstg2, srcs, su23)


def _segsum(tabm01, tabm23, tabu01, tabu23, srcg2, srcs, dstg2, dsts):
    f = functools.partial(
        pl.kernel,
        out_type=[jax.ShapeDtypeStruct((2 * ACC, LANES), jnp.float32)] * 4,
        mesh=_mesh(),
        compiler_params=_SC_PARAMS,
        scratch_types=[pltpu.VMEM((ECHUNK,), jnp.int32),
                       pltpu.VMEM((ECHUNK,), jnp.int32),
                       pltpu.VMEM((ECHUNK, LANES), jnp.float32),
                       pltpu.VMEM((ZROWS, LANES), jnp.float32),
                       pltpu.VMEM_SHARED((ACC, LANES), jnp.float32),
                       pltpu.SemaphoreType.DMA,
                       pltpu.SemaphoreType.DMA,
                       pltpu.SemaphoreType.DMA],
    )(_seg_body)
    return f(tabm01, tabm23, tabu01, tabu23, srcg2, srcs, dstg2, dsts)


def _edge_body(au, am, src_g, dst_g, zsum, idx1_v, idx2_v, buf_v, sem, sem2):
    cid = lax.axis_index("c")
    sid = lax.axis_index("s")
    wid = sid * NC + cid

    def chunk(i, _):
        base = wid * EPW + i * ECH_E
        di = [pltpu.async_copy(src_g.at[pl.ds(base, ECH_E)], idx1_v, sem),
              pltpu.async_copy(dst_g.at[pl.ds(base, ECH_E)], idx2_v, sem2)]
        for d_ in di:
            d_.wait()
        pltpu.async_copy(au.at[idx1_v], buf_v, sem).wait()
        pltpu.async_copy(am.at[idx2_v], buf_v, sem2, add=True).wait()
        pltpu.sync_copy(buf_v, zsum.at[pl.ds(base, ECH_E)])
        return 0
    lax.fori_loop(0, NCH_EDGE, chunk, 0)


def _edge_gather(au, am, src_g2, dst_g2):
    f = functools.partial(
        pl.kernel,
        out_type=jax.ShapeDtypeStruct((E_PAD, H), jnp.float32),
        mesh=_mesh(),
        compiler_params=_SC_PARAMS,
        scratch_types=[pltpu.VMEM((ECH_E,), jnp.int32),
                       pltpu.VMEM((ECH_E,), jnp.int32),
                       pltpu.VMEM((ECH_E, H), jnp.float32),
                       pltpu.SemaphoreType.DMA,
                       pltpu.SemaphoreType.DMA],
    )(_edge_body)
    return f(au, am, src_g2, dst_g2)


def _dense1_body(xu, xm, wl0, wl1, wr0, wr1, b, pm01, pm23, pu01, pu23, rm, ru):
    pm = jnp.dot(xu[...], wl0[...], preferred_element_type=jnp.float32)
    pu = jnp.dot(xm[...], wl1[...], preferred_element_type=jnp.float32)
    pm01[...] = pm[:, :32]
    pm23[...] = pm[:, 32:]
    pu01[...] = pu[:, :32]
    pu23[...] = pu[:, 32:]
    rm[...] = jnp.dot(xm[...], wr0[...], preferred_element_type=jnp.float32) + b[0:1, :]
    ru[...] = jnp.dot(xu[...], wr1[...], preferred_element_type=jnp.float32) + b[1:2, :]


def _dense1(xu, xm, wl0, wl1, wr0, wr1, b):
    nb = N // BN
    row = lambda i: (i, 0)
    full = lambda i: (0, 0)
    return pl.pallas_call(
        _dense1_body,
        grid=(nb,),
        in_specs=[pl.BlockSpec((BN, D), row), pl.BlockSpec((BN, D), row),
                  pl.BlockSpec((D, H), full), pl.BlockSpec((D, H), full),
                  pl.BlockSpec((D, H), full), pl.BlockSpec((D, H), full),
                  pl.BlockSpec((2, H), full)],
        out_specs=[pl.BlockSpec((BN, 32), row)] * 4 +
                  [pl.BlockSpec((BN, H), row)] * 2,
        out_shape=[jax.ShapeDtypeStruct((N, 32), jnp.float32)] * 4 +
                  [jax.ShapeDtypeStruct((N, H), jnp.float32)] * 2,
    )(xu, xm, wl0, wl1, wr0, wr1, b)


_seg_even = lambda i: (i, 0)
_seg_odd = lambda i: (i + ACC_BLK, 0)


def _dense2_body(*refs):
    (s01, s01b, s23, s23b, u01, u01b, u23, u23b,
     cm, cu, rm, ru, wl0, wl1, wr0, wr1, b,
     pm01, pm23, pu01, pu23, rm2, ru2) = refs
    segm = jnp.concatenate([s01[...], s01b[...], s23[...], s23b[...]], axis=1)
    segu = jnp.concatenate([u01[...], u01b[...], u23[...], u23b[...]], axis=1)
    invm = 1.0 / jnp.maximum(cm[:, 0:1], 1.0)
    invu = 1.0 / jnp.maximum(cu[:, 0:1], 1.0)
    hm = jnp.maximum(segm * invm + rm[...], 0.0)
    hu = jnp.maximum(segu * invu + ru[...], 0.0)
    pm = jnp.dot(hu, wl0[...], preferred_element_type=jnp.float32)
    pu = jnp.dot(hm, wl1[...], preferred_element_type=jnp.float32)
    pm01[...] = pm[:, :32]
    pm23[...] = pm[:, 32:]
    pu01[...] = pu[:, :32]
    pu23[...] = pu[:, 32:]
    rm2[...] = jnp.dot(hm, wr0[...], preferred_element_type=jnp.float32) + b[0:1, :]
    ru2[...] = jnp.dot(hu, wr1[...], preferred_element_type=jnp.float32) + b[1:2, :]


def _dense2(sm01, sm23, su01, su23, cnt, rm, ru, wl0, wl1, wr0, wr1, b):
    nb = N // BN
    row = lambda i: (i, 0)
    full = lambda i: (0, 0)
    seg_spec = [pl.BlockSpec((BN, LANES), _seg_even),
                pl.BlockSpec((BN, LANES), _seg_odd)]
    return pl.pallas_call(
        _dense2_body,
        grid=(nb,),
        in_specs=seg_spec * 4 + seg_spec +
                 [pl.BlockSpec((BN, H), row)] * 2 +
                 [pl.BlockSpec((H, H), full)] * 4 +
                 [pl.BlockSpec((2, H), full)],
        out_specs=[pl.BlockSpec((BN, 32), row)] * 4 +
                  [pl.BlockSpec((BN, H), row)] * 2,
        out_shape=[jax.ShapeDtypeStruct((N, 32), jnp.float32)] * 4 +
                  [jax.ShapeDtypeStruct((N, H), jnp.float32)] * 2,
    )(sm01, sm01, sm23, sm23, su01, su01, su23, su23, cnt, cnt,
      rm, ru, wl0, wl1, wr0, wr1, b)


def _dense3_body(*refs):
    (s01, s01b, s23, s23b, u01, u01b, u23, u23b,
     cm, cu, rm2, ru2, w3a, w3b, au, am) = refs
    segm = jnp.concatenate([s01[...], s01b[...], s23[...], s23b[...]], axis=1)
    segu = jnp.concatenate([u01[...], u01b[...], u23[...], u23b[...]], axis=1)
    invm = 1.0 / jnp.maximum(cm[:, 0:1], 1.0)
    invu = 1.0 / jnp.maximum(cu[:, 0:1], 1.0)
    hm2 = segm * invm + rm2[...]
    hu2 = segu * invu + ru2[...]
    au[...] = jnp.dot(hu2, w3a[...], preferred_element_type=jnp.float32)
    am[...] = jnp.dot(hm2, w3b[...], preferred_element_type=jnp.float32)


def _dense3(sm01, sm23, su01, su23, cnt, rm2, ru2, w3a, w3b):
    nb = N // BN
    row = lambda i: (i, 0)
    full = lambda i: (0, 0)
    seg_spec = [pl.BlockSpec((BN, LANES), _seg_even),
                pl.BlockSpec((BN, LANES), _seg_odd)]
    return pl.pallas_call(
        _dense3_body,
        grid=(nb,),
        in_specs=seg_spec * 4 + seg_spec +
                 [pl.BlockSpec((BN, H), row)] * 2 +
                 [pl.BlockSpec((H, H), full)] * 2,
        out_specs=[pl.BlockSpec((BN, H), row)] * 2,
        out_shape=[jax.ShapeDtypeStruct((N, H), jnp.float32)] * 2,
    )(sm01, sm01, sm23, sm23, su01, su01, su23, su23, cnt, cnt,
      rm2, ru2, w3a, w3b)


BE = 2048


def _final_body(z, b3, w4, b4, out):
    h = jnp.maximum(z[...] + b3[0:1, :], 0.0)
    out[...] = jnp.dot(h, w4[...], preferred_element_type=jnp.float32) + b4[0:1, :]


def _final(zsum, b3, w4, b4):
    nb = E_PAD // BE
    row = lambda i: (i, 0)
    full = lambda i: (0, 0)
    return pl.pallas_call(
        _final_body,
        grid=(nb,),
        in_specs=[pl.BlockSpec((BE, H), row), pl.BlockSpec((1, H), full),
                  pl.BlockSpec((H, 2), full), pl.BlockSpec((1, 2), full)],
        out_specs=pl.BlockSpec((BE, 2), row),
        out_shape=jax.ShapeDtypeStruct((E_PAD, 2), jnp.float32),
    )(zsum, b3, w4, b4)


def kernel(x_user, x_merchant, edge_index, Wl1, Wr1, b1, c1,
           Wl2, Wr2, b2, c2, W3, b3, W4, b4):
    src = edge_index[0]
    dst = edge_index[1]
    pad = E_PAD - E
    zpad = jnp.zeros((pad,), jnp.int32)
    spad = jnp.full((pad,), SENT, jnp.int32)
    src_g = jnp.concatenate([src, zpad])
    dst_g = jnp.concatenate([dst, zpad])
    src_gd = jnp.concatenate([src_g * 2, src_g * 2 + 1])
    dst_gd = jnp.concatenate([dst_g * 2, dst_g * 2 + 1])
    src_s2 = jnp.concatenate([src, spad])
    dst_s2 = jnp.concatenate([dst, spad])
    deg_idx = jnp.concatenate([dst_s2, src_s2])

    wl1e = jnp.einsum("ti,idh->tdh", c1, Wl1)
    wr1e = jnp.einsum("ti,idh->tdh", c1, Wr1)
    b1e = c1 @ b1
    wl2e = jnp.einsum("ti,idh->tdh", c2, Wl2)
    wr2e = jnp.einsum("ti,idh->tdh", c2, Wr2)
    b2e = c2 @ b2

    cnt = _degrees(deg_idx)

    pm01, pm23, pu01, pu23, rm, ru = _dense1(
        x_user, x_merchant, wl1e[0], wl1e[1], wr1e[0], wr1e[1], b1e)

    iv = lambda t: t.reshape(2 * N, LANES)
    sm01, sm23, su01, su23 = _segsum(
        iv(pm01), iv(pm23), iv(pu01), iv(pu23),
        src_gd, src_s2, dst_gd, dst_s2)

    pm201, pm223, pu201, pu223, rm2, ru2 = _dense2(
        sm01, sm23, su01, su23, cnt, rm, ru,
        wl2e[0], wl2e[1], wr2e[0], wr2e[1], b2e)

    sm201, sm223, su201, su223 = _segsum(
        iv(pm201), iv(pm223), iv(pu201), iv(pu223),
        src_gd, src_s2, dst_gd, dst_s2)

    au, am = _dense3(sm201, sm223, su201, su223, cnt, rm2, ru2,
                     W3[:H], W3[H:])

    zsum = _edge_gather(au, am, src_g, dst_g)

    outp = _final(zsum, b3.reshape(1, H), W4, b4.reshape(1, 2))
    return outp[:E]


# speedup vs baseline: 1.0013x; 1.0013x over previous
import functools

import jax
import jax.numpy as jnp
from jax import lax
from jax.experimental import pallas as pl
from jax.experimental.pallas import tpu as pltpu
from jax.experimental.pallas import tpu_sc as plsc

N = 50000
E = 600000
D = 128
H = 64
NC = 2
NS = 16
LANES = 16
NW = NC * NS

E_PAD = 622592
EPT_SC = E_PAD // NS
EPW = E_PAD // NW
ECHUNK = 2048
NCH_SEG = EPT_SC // ECHUNK
ECH_E = 1024
NCH_EDGE = EPW // ECH_E

ACC = 51200
SENT = 50000
RPT = ACC // NS
ZROWS = 128
NZCOPY = RPT // ZROWS

BN = 1024
ACC_BLK = ACC // BN


def _mesh():
    return plsc.VectorSubcoreMesh(
        core_axis_name="c", subcore_axis_name="s",
        num_cores=NC, num_subcores=NS)


_SC_PARAMS = pltpu.CompilerParams(use_tc_tiling_on_sc=False)


def _deg_body(idx_hbm, cnt, idx_v, ones_v, zb_v, acc_sh, semi, sem):
    cid = lax.axis_index("c")
    sid = lax.axis_index("s")

    def fill_ones(i, _):
        ones_v[i, :] = jnp.ones((LANES,), jnp.float32)
        return 0
    lax.fori_loop(0, ECHUNK, fill_ones, 0)

    def fill_zeros(i, _):
        zb_v[i, :] = jnp.zeros((LANES,), jnp.float32)
        return 0
    lax.fori_loop(0, ZROWS, fill_zeros, 0)
    dz = [pltpu.async_copy(
              zb_v, acc_sh.at[pl.ds(sid * RPT + i * ZROWS, ZROWS)], semi)
          for i in range(NZCOPY)]
    for d_ in dz:
        d_.wait()
    plsc.subcore_barrier()

    def chunk(i, _):
        base = cid * E_PAD + sid * EPT_SC + i * ECHUNK
        pltpu.async_copy(idx_hbm.at[pl.ds(base, ECHUNK)], idx_v, semi).wait()
        pltpu.async_copy(ones_v, acc_sh.at[idx_v], sem, add=True).wait()
        return 0
    lax.fori_loop(0, NCH_SEG, chunk, 0)
    plsc.subcore_barrier()

    start = sid * RPT
    pltpu.sync_copy(acc_sh.at[pl.ds(start, RPT)],
                    cnt.at[pl.ds(cid * ACC + start, RPT)])


def _degrees(idx_stacked):
    f = functools.partial(
        pl.kernel,
        out_type=jax.ShapeDtypeStruct((2 * ACC, LANES), jnp.float32),
        mesh=_mesh(),
        compiler_params=_SC_PARAMS,
        scratch_types=[pltpu.VMEM((ECHUNK,), jnp.int32),
                       pltpu.VMEM((ECHUNK, LANES), jnp.float32),
                       pltpu.VMEM((ZROWS, LANES), jnp.float32),
                       pltpu.VMEM_SHARED((ACC, LANES), jnp.float32),
                       pltpu.SemaphoreType.DMA,
                       pltpu.SemaphoreType.DMA],
    )(_deg_body)
    return f(idx_stacked)


def _seg_body(tabm01, tabm23, tabu01, tabu23,
              srcg2, srcs, dstg2, dsts,
              sm01, sm23, su01, su23,
              idxg_v, idxs_v, rows_v, zb_v, acc_sh, semi, sem, sem2):
    cid = lax.axis_index("c")
    sid = lax.axis_index("s")

    def fill_zeros(i, _):
        zb_v[i, :] = jnp.zeros((LANES,), jnp.float32)
        return 0
    lax.fori_loop(0, ZROWS, fill_zeros, 0)

    def one_pass(tab, gidx2, sidx, out):
        dz = [pltpu.async_copy(
                  zb_v, acc_sh.at[pl.ds(sid * RPT + i * ZROWS, ZROWS)], semi)
              for i in range(NZCOPY)]
        for d_ in dz:
            d_.wait()
        plsc.subcore_barrier()

        def chunk(i, _):
            gbase = cid * E_PAD + sid * EPT_SC + i * ECHUNK
            sbase = sid * EPT_SC + i * ECHUNK
            di = [pltpu.async_copy(gidx2.at[pl.ds(gbase, ECHUNK)], idxg_v, semi),
                  pltpu.async_copy(sidx.at[pl.ds(sbase, ECHUNK)], idxs_v, semi)]
            for d_ in di:
                d_.wait()
            pltpu.async_copy(tab.at[idxg_v], rows_v, sem).wait()
            pltpu.async_copy(rows_v, acc_sh.at[idxs_v], sem2, add=True).wait()
            return 0
        lax.fori_loop(0, NCH_SEG, chunk, 0)
        plsc.subcore_barrier()

        start = sid * RPT
        pltpu.sync_copy(acc_sh.at[pl.ds(start, RPT)],
                        out.at[pl.ds(cid * ACC + start, RPT)])
        plsc.subcore_barrier()

    one_pass(tabm01, srcg2, dsts, sm01)
    one_pass(tabm23, srcg2, dsts, sm23)
    one_pass(tabu01, dstg2, srcs, su01)
    one_pass(tabu23, dstg2, srcs, su23)


def _segsum(tabm01, tabm23, tabu01, tabu23, srcg2, srcs, dstg2, dsts):
    f = functools.partial(
        pl.kernel,
        out_type=[jax.ShapeDtypeStruct((2 * ACC, LANES), jnp.float32)] * 4,
        mesh=_mesh(),
        compiler_params=_SC_PARAMS,
        scratch_types=[pltpu.VMEM((ECHUNK,), jnp.int32),
                       pltpu.VMEM((ECHUNK,), jnp.int32),
                       pltpu.VMEM((ECHUNK, LANES), jnp.float32),
                       pltpu.VMEM((ZROWS, LANES), jnp.float32),
                       pltpu.VMEM_SHARED((ACC, LANES), jnp.float32),
                       pltpu.SemaphoreType.DMA,
                       pltpu.SemaphoreType.DMA,
                       pltpu.SemaphoreType.DMA],
    )(_seg_body)
    return f(tabm01, tabm23, tabu01, tabu23, srcg2, srcs, dstg2, dsts)


def _edge_body(au, am, src_g, dst_g, zsum, idx1_v, idx2_v, buf_v, sem, sem2):
    cid = lax.axis_index("c")
    sid = lax.axis_index("s")
    wid = sid * NC + cid

    def chunk(i, _):
        base = wid * EPW + i * ECH_E
        di = [pltpu.async_copy(src_g.at[pl.ds(base, ECH_E)], idx1_v, sem),
              pltpu.async_copy(dst_g.at[pl.ds(base, ECH_E)], idx2_v, sem2)]
        for d_ in di:
            d_.wait()
        pltpu.async_copy(au.at[idx1_v], buf_v, sem).wait()
        pltpu.async_copy(am.at[idx2_v], buf_v, sem2, add=True).wait()
        pltpu.sync_copy(buf_v, zsum.at[pl.ds(base, ECH_E)])
        return 0
    lax.fori_loop(0, NCH_EDGE, chunk, 0)


def _edge_gather(au, am, src_g2, dst_g2):
    f = functools.partial(
        pl.kernel,
        out_type=jax.ShapeDtypeStruct((E_PAD, H), jnp.float32),
        mesh=_mesh(),
        compiler_params=_SC_PARAMS,
        scratch_types=[pltpu.VMEM((ECH_E,), jnp.int32),
                       pltpu.VMEM((ECH_E,), jnp.int32),
                       pltpu.VMEM((ECH_E, H), jnp.float32),
                       pltpu.SemaphoreType.DMA,
                       pltpu.SemaphoreType.DMA],
    )(_edge_body)
    return f(au, am, src_g2, dst_g2)


def _dense1_body(xu, xm, wl0, wl1, wr0, wr1, b, pm01, pm23, pu01, pu23, rm, ru):
    pm = jnp.dot(xu[...], wl0[...], preferred_element_type=jnp.float32)
    pu = jnp.dot(xm[...], wl1[...], preferred_element_type=jnp.float32)
    pm01[...] = pm[:, :32]
    pm23[...] = pm[:, 32:]
    pu01[...] = pu[:, :32]
    pu23[...] = pu[:, 32:]
    rm[...] = jnp.dot(xm[...], wr0[...], preferred_element_type=jnp.float32) + b[0:1, :]
    ru[...] = jnp.dot(xu[...], wr1[...], preferred_element_type=jnp.float32) + b[1:2, :]


def _dense1(xu, xm, wl0, wl1, wr0, wr1, b):
    nb = ACC // BN
    row = lambda i: (i, 0)
    full = lambda i: (0, 0)
    return pl.pallas_call(
        _dense1_body,
        grid=(nb,),
        in_specs=[pl.BlockSpec((BN, D), row), pl.BlockSpec((BN, D), row),
                  pl.BlockSpec((D, H), full), pl.BlockSpec((D, H), full),
                  pl.BlockSpec((D, H), full), pl.BlockSpec((D, H), full),
                  pl.BlockSpec((2, H), full)],
        out_specs=[pl.BlockSpec((BN, 32), row)] * 4 +
                  [pl.BlockSpec((BN, H), row)] * 2,
        out_shape=[jax.ShapeDtypeStruct((ACC, 32), jnp.float32)] * 4 +
                  [jax.ShapeDtypeStruct((ACC, H), jnp.float32)] * 2,
    )(xu, xm, wl0, wl1, wr0, wr1, b)


_seg_even = lambda i: (i, 0)
_seg_odd = lambda i: (i + ACC_BLK, 0)


def _dense2_body(*refs):
    (s01, s01b, s23, s23b, u01, u01b, u23, u23b,
     cm, cu, rm, ru, wl0, wl1, wr0, wr1, b,
     pm01, pm23, pu01, pu23, rm2, ru2) = refs
    segm = jnp.concatenate([s01[...], s01b[...], s23[...], s23b[...]], axis=1)
    segu = jnp.concatenate([u01[...], u01b[...], u23[...], u23b[...]], axis=1)
    invm = 1.0 / jnp.maximum(cm[:, 0:1], 1.0)
    invu = 1.0 / jnp.maximum(cu[:, 0:1], 1.0)
    hm = jnp.maximum(segm * invm + rm[...], 0.0)
    hu = jnp.maximum(segu * invu + ru[...], 0.0)
    pm = jnp.dot(hu, wl0[...], preferred_element_type=jnp.float32)
    pu = jnp.dot(hm, wl1[...], preferred_element_type=jnp.float32)
    pm01[...] = pm[:, :32]
    pm23[...] = pm[:, 32:]
    pu01[...] = pu[:, :32]
    pu23[...] = pu[:, 32:]
    rm2[...] = jnp.dot(hm, wr0[...], preferred_element_type=jnp.float32) + b[0:1, :]
    ru2[...] = jnp.dot(hu, wr1[...], preferred_element_type=jnp.float32) + b[1:2, :]


def _dense2(sm01, sm23, su01, su23, cnt, rm, ru, wl0, wl1, wr0, wr1, b):
    nb = ACC // BN
    row = lambda i: (i, 0)
    full = lambda i: (0, 0)
    seg_spec = [pl.BlockSpec((BN, LANES), _seg_even),
                pl.BlockSpec((BN, LANES), _seg_odd)]
    return pl.pallas_call(
        _dense2_body,
        grid=(nb,),
        in_specs=seg_spec * 4 + seg_spec +
                 [pl.BlockSpec((BN, H), row)] * 2 +
                 [pl.BlockSpec((H, H), full)] * 4 +
                 [pl.BlockSpec((2, H), full)],
        out_specs=[pl.BlockSpec((BN, 32), row)] * 4 +
                  [pl.BlockSpec((BN, H), row)] * 2,
        out_shape=[jax.ShapeDtypeStruct((ACC, 32), jnp.float32)] * 4 +
                  [jax.ShapeDtypeStruct((ACC, H), jnp.float32)] * 2,
    )(sm01, sm01, sm23, sm23, su01, su01, su23, su23, cnt, cnt,
      rm, ru, wl0, wl1, wr0, wr1, b)


def _dense3_body(*refs):
    (s01, s01b, s23, s23b, u01, u01b, u23, u23b,
     cm, cu, rm2, ru2, w3a, w3b, au, am) = refs
    segm = jnp.concatenate([s01[...], s01b[...], s23[...], s23b[...]], axis=1)
    segu = jnp.concatenate([u01[...], u01b[...], u23[...], u23b[...]], axis=1)
    invm = 1.0 / jnp.maximum(cm[:, 0:1], 1.0)
    invu = 1.0 / jnp.maximum(cu[:, 0:1], 1.0)
    hm2 = segm * invm + rm2[...]
    hu2 = segu * invu + ru2[...]
    au[...] = jnp.dot(hu2, w3a[...], preferred_element_type=jnp.float32)
    am[...] = jnp.dot(hm2, w3b[...], preferred_element_type=jnp.float32)


def _dense3(sm01, sm23, su01, su23, cnt, rm2, ru2, w3a, w3b):
    nb = ACC // BN
    row = lambda i: (i, 0)
    full = lambda i: (0, 0)
    seg_spec = [pl.BlockSpec((BN, LANES), _seg_even),
                pl.BlockSpec((BN, LANES), _seg_odd)]
    return pl.pallas_call(
        _dense3_body,
        grid=(nb,),
        in_specs=seg_spec * 4 + seg_spec +
                 [pl.BlockSpec((BN, H), row)] * 2 +
                 [pl.BlockSpec((H, H), full)] * 2,
        out_specs=[pl.BlockSpec((BN, H), row)] * 2,
        out_shape=[jax.ShapeDtypeStruct((ACC, H), jnp.float32)] * 2,
    )(sm01, sm01, sm23, sm23, su01, su01, su23, su23, cnt, cnt,
      rm2, ru2, w3a, w3b)


BE = 2048


def _final_body(z, b3, w4, b4, out):
    h = jnp.maximum(z[...] + b3[0:1, :], 0.0)
    out[...] = jnp.dot(h, w4[...], preferred_element_type=jnp.float32) + b4[0:1, :]


def _final(zsum, b3, w4, b4):
    nb = E_PAD // BE
    row = lambda i: (i, 0)
    full = lambda i: (0, 0)
    return pl.pallas_call(
        _final_body,
        grid=(nb,),
        in_specs=[pl.BlockSpec((BE, H), row), pl.BlockSpec((1, H), full),
                  pl.BlockSpec((H, 2), full), pl.BlockSpec((1, 2), full)],
        out_specs=pl.BlockSpec((BE, 2), row),
        out_shape=jax.ShapeDtypeStruct((E_PAD, 2), jnp.float32),
    )(zsum, b3, w4, b4)


def kernel(x_user, x_merchant, edge_index, Wl1, Wr1, b1, c1,
           Wl2, Wr2, b2, c2, W3, b3, W4, b4):
    src = edge_index[0]
    dst = edge_index[1]
    pad = E_PAD - E
    zpad = jnp.zeros((pad,), jnp.int32)
    spad = jnp.full((pad,), SENT, jnp.int32)
    src_g = jnp.concatenate([src, zpad])
    dst_g = jnp.concatenate([dst, zpad])
    src_gd = jnp.concatenate([src_g * 2, src_g * 2 + 1])
    dst_gd = jnp.concatenate([dst_g * 2, dst_g * 2 + 1])
    src_s2 = jnp.concatenate([src, spad])
    dst_s2 = jnp.concatenate([dst, spad])
    deg_idx = jnp.concatenate([dst_s2, src_s2])

    wl1e = jnp.einsum("ti,idh->tdh", c1, Wl1)
    wr1e = jnp.einsum("ti,idh->tdh", c1, Wr1)
    b1e = c1 @ b1
    wl2e = jnp.einsum("ti,idh->tdh", c2, Wl2)
    wr2e = jnp.einsum("ti,idh->tdh", c2, Wr2)
    b2e = c2 @ b2

    cnt = _degrees(deg_idx)

    npad = jnp.zeros((ACC - N, D), jnp.float32)
    pm01, pm23, pu01, pu23, rm, ru = _dense1(
        jnp.concatenate([x_user, npad]), jnp.concatenate([x_merchant, npad]),
        wl1e[0], wl1e[1], wr1e[0], wr1e[1], b1e)

    iv = lambda t: t.reshape(2 * ACC, LANES)
    sm01, sm23, su01, su23 = _segsum(
        iv(pm01), iv(pm23), iv(pu01), iv(pu23),
        src_gd, src_s2, dst_gd, dst_s2)

    pm201, pm223, pu201, pu223, rm2, ru2 = _dense2(
        sm01, sm23, su01, su23, cnt, rm, ru,
        wl2e[0], wl2e[1], wr2e[0], wr2e[1], b2e)

    sm201, sm223, su201, su223 = _segsum(
        iv(pm201), iv(pm223), iv(pu201), iv(pu223),
        src_gd, src_s2, dst_gd, dst_s2)

    au, am = _dense3(sm201, sm223, su201, su223, cnt, rm2, ru2,
                     W3[:H], W3[H:])

    zsum = _edge_gather(au, am, src_g, dst_g)

    outp = _final(zsum, b3.reshape(1, H), W4, b4.reshape(1, 2))
    return outp[:E]


# speedup vs baseline: 1.0169x; 1.0156x over previous
import functools

import jax
import jax.numpy as jnp
from jax import lax
from jax.experimental import pallas as pl
from jax.experimental.pallas import tpu as pltpu
from jax.experimental.pallas import tpu_sc as plsc

N = 50000
E = 600000
D = 128
H = 64
NC = 2
NS = 16
LANES = 16
NW = NC * NS

E_PAD = 622592
EPT_SC = E_PAD // NS
EPW = E_PAD // NW
ECHUNK = 2048
NCH_SEG = EPT_SC // ECHUNK
ECH_E = 1024
NCH_EDGE = EPW // ECH_E

ACC = 51200
SENT = 50000
RPT = ACC // NS
ZROWS = 128
NZCOPY = RPT // ZROWS

BN = 1024
ACC_BLK = ACC // BN


def _mesh():
    return plsc.VectorSubcoreMesh(
        core_axis_name="c", subcore_axis_name="s",
        num_cores=NC, num_subcores=NS)


_SC_PARAMS = pltpu.CompilerParams(use_tc_tiling_on_sc=False)


def _deg_body(idx_hbm, cnt, idx_v, ones_v, zb_v, acc_sh, semi, sem):
    cid = lax.axis_index("c")
    sid = lax.axis_index("s")

    def fill_ones(i, _):
        ones_v[i, :] = jnp.ones((LANES,), jnp.float32)
        return 0
    lax.fori_loop(0, ECHUNK, fill_ones, 0)

    def fill_zeros(i, _):
        zb_v[i, :] = jnp.zeros((LANES,), jnp.float32)
        return 0
    lax.fori_loop(0, ZROWS, fill_zeros, 0)
    dz = [pltpu.async_copy(
              zb_v, acc_sh.at[pl.ds(sid * RPT + i * ZROWS, ZROWS)], semi)
          for i in range(NZCOPY)]
    for d_ in dz:
        d_.wait()
    plsc.subcore_barrier()

    def chunk(i, _):
        base = cid * E_PAD + sid * EPT_SC + i * ECHUNK
        pltpu.async_copy(idx_hbm.at[pl.ds(base, ECHUNK)], idx_v, semi).wait()
        pltpu.async_copy(ones_v, acc_sh.at[idx_v], sem, add=True).wait()
        return 0
    lax.fori_loop(0, NCH_SEG, chunk, 0)
    plsc.subcore_barrier()

    start = sid * RPT
    pltpu.sync_copy(acc_sh.at[pl.ds(start, RPT)],
                    cnt.at[pl.ds(cid * ACC + start, RPT)])


def _degrees(idx_stacked):
    f = functools.partial(
        pl.kernel,
        out_type=jax.ShapeDtypeStruct((2 * ACC, LANES), jnp.float32),
        mesh=_mesh(),
        compiler_params=_SC_PARAMS,
        scratch_types=[pltpu.VMEM((ECHUNK,), jnp.int32),
                       pltpu.VMEM((ECHUNK, LANES), jnp.float32),
                       pltpu.VMEM((ZROWS, LANES), jnp.float32),
                       pltpu.VMEM_SHARED((ACC, LANES), jnp.float32),
                       pltpu.SemaphoreType.DMA,
                       pltpu.SemaphoreType.DMA],
    )(_deg_body)
    return f(idx_stacked)


def _seg_body(tabm01, tabm23, tabu01, tabu23,
              srcg2, srcs, dstg2, dsts,
              sm01, sm23, su01, su23,
              idxg_v, idxs_v, rows_v, idxg_b, idxs_b, rows_b,
              zb_v, acc_sh, semi, sem, semb, sem2):
    cid = lax.axis_index("c")
    sid = lax.axis_index("s")

    def fill_zeros(i, _):
        zb_v[i, :] = jnp.zeros((LANES,), jnp.float32)
        return 0
    lax.fori_loop(0, ZROWS, fill_zeros, 0)

    def one_pass(tab, gidx2, sidx, out):
        dz = [pltpu.async_copy(
                  zb_v, acc_sh.at[pl.ds(sid * RPT + i * ZROWS, ZROWS)], semi)
              for i in range(NZCOPY)]
        for d_ in dz:
            d_.wait()
        plsc.subcore_barrier()

        def load_idx(k, ig, is_):
            gbase = cid * E_PAD + sid * EPT_SC + k * ECHUNK
            sbase = sid * EPT_SC + k * ECHUNK
            return [pltpu.async_copy(gidx2.at[pl.ds(gbase, ECHUNK)], ig, semi),
                    pltpu.async_copy(sidx.at[pl.ds(sbase, ECHUNK)], is_, semi)]

        def chunk2(i, _):
            a = 2 * i
            dA = load_idx(a, idxg_v, idxs_v)
            for d_ in dA:
                d_.wait()
            gA = pltpu.async_copy(tab.at[idxg_v], rows_v, sem)
            dB = load_idx(a + 1, idxg_b, idxs_b)
            gA.wait()
            sA = pltpu.async_copy(rows_v, acc_sh.at[idxs_v], sem2, add=True)
            for d_ in dB:
                d_.wait()
            gB = pltpu.async_copy(tab.at[idxg_b], rows_b, semb)
            sA.wait()
            gB.wait()
            pltpu.async_copy(rows_b, acc_sh.at[idxs_b], sem2, add=True).wait()
            return 0
        lax.fori_loop(0, NCH_SEG // 2, chunk2, 0)
        dT = load_idx(NCH_SEG - 1, idxg_v, idxs_v)
        for d_ in dT:
            d_.wait()
        pltpu.async_copy(tab.at[idxg_v], rows_v, sem).wait()
        pltpu.async_copy(rows_v, acc_sh.at[idxs_v], sem2, add=True).wait()
        plsc.subcore_barrier()

        start = sid * RPT
        pltpu.sync_copy(acc_sh.at[pl.ds(start, RPT)],
                        out.at[pl.ds(cid * ACC + start, RPT)])
        plsc.subcore_barrier()

    one_pass(tabm01, srcg2, dsts, sm01)
    one_pass(tabm23, srcg2, dsts, sm23)
    one_pass(tabu01, dstg2, srcs, su01)
    one_pass(tabu23, dstg2, srcs, su23)


def _segsum(tabm01, tabm23, tabu01, tabu23, srcg2, srcs, dstg2, dsts):
    f = functools.partial(
        pl.kernel,
        out_type=[jax.ShapeDtypeStruct((2 * ACC, LANES), jnp.float32)] * 4,
        mesh=_mesh(),
        compiler_params=_SC_PARAMS,
        scratch_types=[pltpu.VMEM((ECHUNK,), jnp.int32),
                       pltpu.VMEM((ECHUNK,), jnp.int32),
                       pltpu.VMEM((ECHUNK, LANES), jnp.float32),
                       pltpu.VMEM((ECHUNK,), jnp.int32),
                       pltpu.VMEM((ECHUNK,), jnp.int32),
                       pltpu.VMEM((ECHUNK, LANES), jnp.float32),
                       pltpu.VMEM((ZROWS, LANES), jnp.float32),
                       pltpu.VMEM_SHARED((ACC, LANES), jnp.float32),
                       pltpu.SemaphoreType.DMA,
                       pltpu.SemaphoreType.DMA,
                       pltpu.SemaphoreType.DMA,
                       pltpu.SemaphoreType.DMA],
    )(_seg_body)
    return f(tabm01, tabm23, tabu01, tabu23, srcg2, srcs, dstg2, dsts)


def _edge_body(au, am, src_g, dst_g, zsum, idx1_v, idx2_v, buf_v, sem, sem2):
    cid = lax.axis_index("c")
    sid = lax.axis_index("s")
    wid = sid * NC + cid

    def chunk(i, _):
        base = wid * EPW + i * ECH_E
        di = [pltpu.async_copy(src_g.at[pl.ds(base, ECH_E)], idx1_v, sem),
              pltpu.async_copy(dst_g.at[pl.ds(base, ECH_E)], idx2_v, sem2)]
        for d_ in di:
            d_.wait()
        pltpu.async_copy(au.at[idx1_v], buf_v, sem).wait()
        pltpu.async_copy(am.at[idx2_v], buf_v, sem2, add=True).wait()
        pltpu.sync_copy(buf_v, zsum.at[pl.ds(base, ECH_E)])
        return 0
    lax.fori_loop(0, NCH_EDGE, chunk, 0)


def _edge_gather(au, am, src_g2, dst_g2):
    f = functools.partial(
        pl.kernel,
        out_type=jax.ShapeDtypeStruct((E_PAD, H), jnp.float32),
        mesh=_mesh(),
        compiler_params=_SC_PARAMS,
        scratch_types=[pltpu.VMEM((ECH_E,), jnp.int32),
                       pltpu.VMEM((ECH_E,), jnp.int32),
                       pltpu.VMEM((ECH_E, H), jnp.float32),
                       pltpu.SemaphoreType.DMA,
                       pltpu.SemaphoreType.DMA],
    )(_edge_body)
    return f(au, am, src_g2, dst_g2)


def _dense1_body(xu, xm, wl0, wl1, wr0, wr1, b, pm01, pm23, pu01, pu23, rm, ru):
    pm = jnp.dot(xu[...], wl0[...], preferred_element_type=jnp.float32)
    pu = jnp.dot(xm[...], wl1[...], preferred_element_type=jnp.float32)
    pm01[...] = pm[:, :32]
    pm23[...] = pm[:, 32:]
    pu01[...] = pu[:, :32]
    pu23[...] = pu[:, 32:]
    rm[...] = jnp.dot(xm[...], wr0[...], preferred_element_type=jnp.float32) + b[0:1, :]
    ru[...] = jnp.dot(xu[...], wr1[...], preferred_element_type=jnp.float32) + b[1:2, :]


def _dense1(xu, xm, wl0, wl1, wr0, wr1, b):
    nb = ACC // BN
    row = lambda i: (i, 0)
    full = lambda i: (0, 0)
    return pl.pallas_call(
        _dense1_body,
        grid=(nb,),
        in_specs=[pl.BlockSpec((BN, D), row), pl.BlockSpec((BN, D), row),
                  pl.BlockSpec((D, H), full), pl.BlockSpec((D, H), full),
                  pl.BlockSpec((D, H), full), pl.BlockSpec((D, H), full),
                  pl.BlockSpec((2, H), full)],
        out_specs=[pl.BlockSpec((BN, 32), row)] * 4 +
                  [pl.BlockSpec((BN, H), row)] * 2,
        out_shape=[jax.ShapeDtypeStruct((ACC, 32), jnp.float32)] * 4 +
                  [jax.ShapeDtypeStruct((ACC, H), jnp.float32)] * 2,
    )(xu, xm, wl0, wl1, wr0, wr1, b)


_seg_even = lambda i: (i, 0)
_seg_odd = lambda i: (i + ACC_BLK, 0)


def _dense2_body(*refs):
    (s01, s01b, s23, s23b, u01, u01b, u23, u23b,
     cm, cu, rm, ru, wl0, wl1, wr0, wr1, b,
     pm01, pm23, pu01, pu23, rm2, ru2) = refs
    segm = jnp.concatenate([s01[...], s01b[...], s23[...], s23b[...]], axis=1)
    segu = jnp.concatenate([u01[...], u01b[...], u23[...], u23b[...]], axis=1)
    invm = 1.0 / jnp.maximum(cm[:, 0:1], 1.0)
    invu = 1.0 / jnp.maximum(cu[:, 0:1], 1.0)
    hm = jnp.maximum(segm * invm + rm[...], 0.0)
    hu = jnp.maximum(segu * invu + ru[...], 0.0)
    pm = jnp.dot(hu, wl0[...], preferred_element_type=jnp.float32)
    pu = jnp.dot(hm, wl1[...], preferred_element_type=jnp.float32)
    pm01[...] = pm[:, :32]
    pm23[...] = pm[:, 32:]
    pu01[...] = pu[:, :32]
    pu23[...] = pu[:, 32:]
    rm2[...] = jnp.dot(hm, wr0[...], preferred_element_type=jnp.float32) + b[0:1, :]
    ru2[...] = jnp.dot(hu, wr1[...], preferred_element_type=jnp.float32) + b[1:2, :]


def _dense2(sm01, sm23, su01, su23, cnt, rm, ru, wl0, wl1, wr0, wr1, b):
    nb = ACC // BN
    row = lambda i: (i, 0)
    full = lambda i: (0, 0)
    seg_spec = [pl.BlockSpec((BN, LANES), _seg_even),
                pl.BlockSpec((BN, LANES), _seg_odd)]
    return pl.pallas_call(
        _dense2_body,
        grid=(nb,),
        in_specs=seg_spec * 4 + seg_spec +
                 [pl.BlockSpec((BN, H), row)] * 2 +
                 [pl.BlockSpec((H, H), full)] * 4 +
                 [pl.BlockSpec((2, H), full)],
        out_specs=[pl.BlockSpec((BN, 32), row)] * 4 +
                  [pl.BlockSpec((BN, H), row)] * 2,
        out_shape=[jax.ShapeDtypeStruct((ACC, 32), jnp.float32)] * 4 +
                  [jax.ShapeDtypeStruct((ACC, H), jnp.float32)] * 2,
    )(sm01, sm01, sm23, sm23, su01, su01, su23, su23, cnt, cnt,
      rm, ru, wl0, wl1, wr0, wr1, b)


def _dense3_body(*refs):
    (s01, s01b, s23, s23b, u01, u01b, u23, u23b,
     cm, cu, rm2, ru2, w3a, w3b, au, am) = refs
    segm = jnp.concatenate([s01[...], s01b[...], s23[...], s23b[...]], axis=1)
    segu = jnp.concatenate([u01[...], u01b[...], u23[...], u23b[...]], axis=1)
    invm = 1.0 / jnp.maximum(cm[:, 0:1], 1.0)
    invu = 1.0 / jnp.maximum(cu[:, 0:1], 1.0)
    hm2 = segm * invm + rm2[...]
    hu2 = segu * invu + ru2[...]
    au[...] = jnp.dot(hu2, w3a[...], preferred_element_type=jnp.float32)
    am[...] = jnp.dot(hm2, w3b[...], preferred_element_type=jnp.float32)


def _dense3(sm01, sm23, su01, su23, cnt, rm2, ru2, w3a, w3b):
    nb = ACC // BN
    row = lambda i: (i, 0)
    full = lambda i: (0, 0)
    seg_spec = [pl.BlockSpec((BN, LANES), _seg_even),
                pl.BlockSpec((BN, LANES), _seg_odd)]
    return pl.pallas_call(
        _dense3_body,
        grid=(nb,),
        in_specs=seg_spec * 4 + seg_spec +
                 [pl.BlockSpec((BN, H), row)] * 2 +
                 [pl.BlockSpec((H, H), full)] * 2,
        out_specs=[pl.BlockSpec((BN, H), row)] * 2,
        out_shape=[jax.ShapeDtypeStruct((ACC, H), jnp.float32)] * 2,
    )(sm01, sm01, sm23, sm23, su01, su01, su23, su23, cnt, cnt,
      rm2, ru2, w3a, w3b)


BE = 2048


def _final_body(z, b3, w4, b4, out):
    h = jnp.maximum(z[...] + b3[0:1, :], 0.0)
    out[...] = jnp.dot(h, w4[...], preferred_element_type=jnp.float32) + b4[0:1, :]


def _final(zsum, b3, w4, b4):
    nb = E_PAD // BE
    row = lambda i: (i, 0)
    full = lambda i: (0, 0)
    return pl.pallas_call(
        _final_body,
        grid=(nb,),
        in_specs=[pl.BlockSpec((BE, H), row), pl.BlockSpec((1, H), full),
                  pl.BlockSpec((H, 2), full), pl.BlockSpec((1, 2), full)],
        out_specs=pl.BlockSpec((BE, 2), row),
        out_shape=jax.ShapeDtypeStruct((E_PAD, 2), jnp.float32),
    )(zsum, b3, w4, b4)


def kernel(x_user, x_merchant, edge_index, Wl1, Wr1, b1, c1,
           Wl2, Wr2, b2, c2, W3, b3, W4, b4):
    src = edge_index[0]
    dst = edge_index[1]
    pad = E_PAD - E
    zpad = jnp.zeros((pad,), jnp.int32)
    spad = jnp.full((pad,), SENT, jnp.int32)
    src_g = jnp.concatenate([src, zpad])
    dst_g = jnp.concatenate([dst, zpad])
    src_gd = jnp.concatenate([src_g * 2, src_g * 2 + 1])
    dst_gd = jnp.concatenate([dst_g * 2, dst_g * 2 + 1])
    src_s2 = jnp.concatenate([src, spad])
    dst_s2 = jnp.concatenate([dst, spad])
    deg_idx = jnp.concatenate([dst_s2, src_s2])

    wl1e = jnp.einsum("ti,idh->tdh", c1, Wl1)
    wr1e = jnp.einsum("ti,idh->tdh", c1, Wr1)
    b1e = c1 @ b1
    wl2e = jnp.einsum("ti,idh->tdh", c2, Wl2)
    wr2e = jnp.einsum("ti,idh->tdh", c2, Wr2)
    b2e = c2 @ b2

    cnt = _degrees(deg_idx)

    npad = jnp.zeros((ACC - N, D), jnp.float32)
    pm01, pm23, pu01, pu23, rm, ru = _dense1(
        jnp.concatenate([x_user, npad]), jnp.concatenate([x_merchant, npad]),
        wl1e[0], wl1e[1], wr1e[0], wr1e[1], b1e)

    iv = lambda t: t.reshape(2 * ACC, LANES)
    sm01, sm23, su01, su23 = _segsum(
        iv(pm01), iv(pm23), iv(pu01), iv(pu23),
        src_gd, src_s2, dst_gd, dst_s2)

    pm201, pm223, pu201, pu223, rm2, ru2 = _dense2(
        sm01, sm23, su01, su23, cnt, rm, ru,
        wl2e[0], wl2e[1], wr2e[0], wr2e[1], b2e)

    sm201, sm223, su201, su223 = _segsum(
        iv(pm201), iv(pm223), iv(pu201), iv(pu223),
        src_gd, src_s2, dst_gd, dst_s2)

    au, am = _dense3(sm201, sm223, su201, su223, cnt, rm2, ru2,
                     W3[:H], W3[H:])

    zsum = _edge_gather(au, am, src_g, dst_g)

    outp = _final(zsum, b3.reshape(1, H), W4, b4.reshape(1, 2))
    return outp[:E]


# speedup vs baseline: 1.0595x; 1.0419x over previous
import functools

import jax
import jax.numpy as jnp
from jax import lax
from jax.experimental import pallas as pl
from jax.experimental.pallas import tpu as pltpu
from jax.experimental.pallas import tpu_sc as plsc

N = 50000
E = 600000
D = 128
H = 64
NC = 2
NS = 16
LANES = 16
NW = NC * NS

E_PAD = 622592
EPT_SC = E_PAD // NS
EPW = E_PAD // NW
ECHUNK = 2048
NCH_DEG2 = EPT_SC // ECHUNK
ECH_S = 512
NCH_SEG = EPT_SC // ECH_S
ECH_E = 1024
NCH_EDGE = EPW // ECH_E

ACC = 51200
SENT = 50000
RPT = ACC // NS
ZROWS = 128
NZCOPY = RPT // ZROWS

BN = 1024
ACC_BLK = ACC // BN


def _mesh():
    return plsc.VectorSubcoreMesh(
        core_axis_name="c", subcore_axis_name="s",
        num_cores=NC, num_subcores=NS)


_SC_PARAMS = pltpu.CompilerParams(use_tc_tiling_on_sc=False)


def _deg_body(idx_hbm, cnt, idx_v, ones_v, zb_v, acc_sh, semi, sem):
    cid = lax.axis_index("c")
    sid = lax.axis_index("s")

    def fill_ones(i, _):
        ones_v[i, :] = jnp.ones((LANES,), jnp.float32)
        return 0
    lax.fori_loop(0, ECHUNK, fill_ones, 0)

    def fill_zeros(i, _):
        zb_v[i, :] = jnp.zeros((LANES,), jnp.float32)
        return 0
    lax.fori_loop(0, ZROWS, fill_zeros, 0)
    dz = [pltpu.async_copy(
              zb_v, acc_sh.at[pl.ds(sid * RPT + i * ZROWS, ZROWS)], semi)
          for i in range(NZCOPY)]
    for d_ in dz:
        d_.wait()
    plsc.subcore_barrier()

    def chunk(i, _):
        base = cid * E_PAD + sid * EPT_SC + i * ECHUNK
        pltpu.async_copy(idx_hbm.at[pl.ds(base, ECHUNK)], idx_v, semi).wait()
        pltpu.async_copy(ones_v, acc_sh.at[idx_v], sem, add=True).wait()
        return 0
    lax.fori_loop(0, NCH_DEG2, chunk, 0)
    plsc.subcore_barrier()

    start = sid * RPT
    pltpu.sync_copy(acc_sh.at[pl.ds(start, RPT)],
                    cnt.at[pl.ds(cid * ACC + start, RPT)])


def _degrees(idx_stacked):
    f = functools.partial(
        pl.kernel,
        out_type=jax.ShapeDtypeStruct((2 * ACC, LANES), jnp.float32),
        mesh=_mesh(),
        compiler_params=_SC_PARAMS,
        scratch_types=[pltpu.VMEM((ECHUNK,), jnp.int32),
                       pltpu.VMEM((ECHUNK, LANES), jnp.float32),
                       pltpu.VMEM((ZROWS, LANES), jnp.float32),
                       pltpu.VMEM_SHARED((ACC, LANES), jnp.float32),
                       pltpu.SemaphoreType.DMA,
                       pltpu.SemaphoreType.DMA],
    )(_deg_body)
    return f(idx_stacked)


def _seg_body(tabm, tabu, srcg2, srcs, dstg2, dsts, sm, su,
              idxg_v, idxs_v, rows_v, zb_v, acc_sh, semi, sem, sem2):
    cid = lax.axis_index("c")
    sid = lax.axis_index("s")

    def fill_zeros(i, _):
        z = jnp.zeros((LANES,), jnp.float32)
        zb_v[i, pl.ds(0, LANES)] = z
        zb_v[i, pl.ds(LANES, LANES)] = z
        return 0
    lax.fori_loop(0, ZROWS, fill_zeros, 0)

    def one_pass(tab, gidx2, sidx, out):
        dz = [pltpu.async_copy(
                  zb_v, acc_sh.at[pl.ds(sid * RPT + i * ZROWS, ZROWS)], semi)
              for i in range(NZCOPY)]
        for d_ in dz:
            d_.wait()
        plsc.subcore_barrier()

        def chunk(i, _):
            gbase = cid * E_PAD + sid * EPT_SC + i * ECH_S
            sbase = sid * EPT_SC + i * ECH_S
            di = [pltpu.async_copy(gidx2.at[pl.ds(gbase, ECH_S)], idxg_v, semi),
                  pltpu.async_copy(sidx.at[pl.ds(sbase, ECH_S)], idxs_v, semi)]
            for d_ in di:
                d_.wait()
            pltpu.async_copy(tab.at[idxg_v], rows_v, sem).wait()
            pltpu.async_copy(rows_v, acc_sh.at[idxs_v], sem2, add=True).wait()
            return 0
        lax.fori_loop(0, NCH_SEG, chunk, 0)
        plsc.subcore_barrier()

        start = sid * RPT
        pltpu.sync_copy(acc_sh.at[pl.ds(start, RPT)],
                        out.at[pl.ds(cid * ACC + start, RPT)])
        plsc.subcore_barrier()

    one_pass(tabm, srcg2, dsts, sm)
    one_pass(tabu, dstg2, srcs, su)


def _segsum(tabm, tabu, srcg2, srcs, dstg2, dsts):
    f = functools.partial(
        pl.kernel,
        out_type=[jax.ShapeDtypeStruct((2 * ACC, 32), jnp.float32)] * 2,
        mesh=_mesh(),
        compiler_params=_SC_PARAMS,
        scratch_types=[pltpu.VMEM((ECH_S,), jnp.int32),
                       pltpu.VMEM((ECH_S,), jnp.int32),
                       pltpu.VMEM((ECH_S, 32), jnp.float32),
                       pltpu.VMEM((ZROWS, 32), jnp.float32),
                       pltpu.VMEM_SHARED((ACC, 32), jnp.float32),
                       pltpu.SemaphoreType.DMA,
                       pltpu.SemaphoreType.DMA,
                       pltpu.SemaphoreType.DMA],
    )(_seg_body)
    return f(tabm, tabu, srcg2, srcs, dstg2, dsts)


def _edge_body(au, am, src_g, dst_g, zsum, idx1_v, idx2_v, buf_v, sem, sem2):
    cid = lax.axis_index("c")
    sid = lax.axis_index("s")
    wid = sid * NC + cid

    def chunk(i, _):
        base = wid * EPW + i * ECH_E
        di = [pltpu.async_copy(src_g.at[pl.ds(base, ECH_E)], idx1_v, sem),
              pltpu.async_copy(dst_g.at[pl.ds(base, ECH_E)], idx2_v, sem2)]
        for d_ in di:
            d_.wait()
        pltpu.async_copy(au.at[idx1_v], buf_v, sem).wait()
        pltpu.async_copy(am.at[idx2_v], buf_v, sem2, add=True).wait()
        pltpu.sync_copy(buf_v, zsum.at[pl.ds(base, ECH_E)])
        return 0
    lax.fori_loop(0, NCH_EDGE, chunk, 0)


def _edge_gather(au, am, src_g2, dst_g2):
    f = functools.partial(
        pl.kernel,
        out_type=jax.ShapeDtypeStruct((E_PAD, H), jnp.float32),
        mesh=_mesh(),
        compiler_params=_SC_PARAMS,
        scratch_types=[pltpu.VMEM((ECH_E,), jnp.int32),
                       pltpu.VMEM((ECH_E,), jnp.int32),
                       pltpu.VMEM((ECH_E, H), jnp.float32),
                       pltpu.SemaphoreType.DMA,
                       pltpu.SemaphoreType.DMA],
    )(_edge_body)
    return f(au, am, src_g2, dst_g2)


def _dense1_body(xu, xm, wl0, wl1, wr0, wr1, b, pm, pu, rm, ru):
    pm[...] = jnp.dot(xu[...], wl0[...], preferred_element_type=jnp.float32)
    pu[...] = jnp.dot(xm[...], wl1[...], preferred_element_type=jnp.float32)
    rm[...] = jnp.dot(xm[...], wr0[...], preferred_element_type=jnp.float32) + b[0:1, :]
    ru[...] = jnp.dot(xu[...], wr1[...], preferred_element_type=jnp.float32) + b[1:2, :]


def _dense1(xu, xm, wl0, wl1, wr0, wr1, b):
    nb = ACC // BN
    row = lambda i: (i, 0)
    full = lambda i: (0, 0)
    return pl.pallas_call(
        _dense1_body,
        grid=(nb,),
        in_specs=[pl.BlockSpec((BN, D), row), pl.BlockSpec((BN, D), row),
                  pl.BlockSpec((D, H), full), pl.BlockSpec((D, H), full),
                  pl.BlockSpec((D, H), full), pl.BlockSpec((D, H), full),
                  pl.BlockSpec((2, H), full)],
        out_specs=[pl.BlockSpec((BN, H), row)] * 4,
        out_shape=[jax.ShapeDtypeStruct((ACC, H), jnp.float32)] * 4,
    )(xu, xm, wl0, wl1, wr0, wr1, b)


_seg_even = lambda i: (i, 0)
_seg_odd = lambda i: (i + ACC_BLK, 0)


def _dense2_body(*refs):
    (sme, smo, sue, suo, cm, cu, rm, ru, wl0, wl1, wr0, wr1, b,
     pm2, pu2, rm2, ru2) = refs
    segm = jnp.concatenate([sme[...], smo[...]], axis=1)
    segu = jnp.concatenate([sue[...], suo[...]], axis=1)
    invm = 1.0 / jnp.maximum(cm[:, 0:1], 1.0)
    invu = 1.0 / jnp.maximum(cu[:, 0:1], 1.0)
    hm = jnp.maximum(segm * invm + rm[...], 0.0)
    hu = jnp.maximum(segu * invu + ru[...], 0.0)
    pm2[...] = jnp.dot(hu, wl0[...], preferred_element_type=jnp.float32)
    pu2[...] = jnp.dot(hm, wl1[...], preferred_element_type=jnp.float32)
    rm2[...] = jnp.dot(hm, wr0[...], preferred_element_type=jnp.float32) + b[0:1, :]
    ru2[...] = jnp.dot(hu, wr1[...], preferred_element_type=jnp.float32) + b[1:2, :]


def _dense2(sm, su, cnt, rm, ru, wl0, wl1, wr0, wr1, b):
    nb = ACC // BN
    row = lambda i: (i, 0)
    full = lambda i: (0, 0)
    seg_spec = [pl.BlockSpec((BN, 32), _seg_even),
                pl.BlockSpec((BN, 32), _seg_odd)]
    cnt_spec = [pl.BlockSpec((BN, LANES), _seg_even),
                pl.BlockSpec((BN, LANES), _seg_odd)]
    return pl.pallas_call(
        _dense2_body,
        grid=(nb,),
        in_specs=seg_spec * 2 + cnt_spec +
                 [pl.BlockSpec((BN, H), row)] * 2 +
                 [pl.BlockSpec((H, H), full)] * 4 +
                 [pl.BlockSpec((2, H), full)],
        out_specs=[pl.BlockSpec((BN, H), row)] * 4,
        out_shape=[jax.ShapeDtypeStruct((ACC, H), jnp.float32)] * 4,
    )(sm, sm, su, su, cnt, cnt, rm, ru, wl0, wl1, wr0, wr1, b)


def _dense3_body(*refs):
    (sme, smo, sue, suo, cm, cu, rm2, ru2, w3a, w3b, au, am) = refs
    segm = jnp.concatenate([sme[...], smo[...]], axis=1)
    segu = jnp.concatenate([sue[...], suo[...]], axis=1)
    invm = 1.0 / jnp.maximum(cm[:, 0:1], 1.0)
    invu = 1.0 / jnp.maximum(cu[:, 0:1], 1.0)
    hm2 = segm * invm + rm2[...]
    hu2 = segu * invu + ru2[...]
    au[...] = jnp.dot(hu2, w3a[...], preferred_element_type=jnp.float32)
    am[...] = jnp.dot(hm2, w3b[...], preferred_element_type=jnp.float32)


def _dense3(sm, su, cnt, rm2, ru2, w3a, w3b):
    nb = ACC // BN
    row = lambda i: (i, 0)
    full = lambda i: (0, 0)
    seg_spec = [pl.BlockSpec((BN, 32), _seg_even),
                pl.BlockSpec((BN, 32), _seg_odd)]
    cnt_spec = [pl.BlockSpec((BN, LANES), _seg_even),
                pl.BlockSpec((BN, LANES), _seg_odd)]
    return pl.pallas_call(
        _dense3_body,
        grid=(nb,),
        in_specs=seg_spec * 2 + cnt_spec +
                 [pl.BlockSpec((BN, H), row)] * 2 +
                 [pl.BlockSpec((H, H), full)] * 2,
        out_specs=[pl.BlockSpec((BN, H), row)] * 2,
        out_shape=[jax.ShapeDtypeStruct((ACC, H), jnp.float32)] * 2,
    )(sm, sm, su, su, cnt, cnt, rm2, ru2, w3a, w3b)


BE = 2048


def _final_body(z, b3, w4, b4, out):
    h = jnp.maximum(z[...] + b3[0:1, :], 0.0)
    out[...] = jnp.dot(h, w4[...], preferred_element_type=jnp.float32) + b4[0:1, :]


def _final(zsum, b3, w4, b4):
    nb = E_PAD // BE
    row = lambda i: (i, 0)
    full = lambda i: (0, 0)
    return pl.pallas_call(
        _final_body,
        grid=(nb,),
        in_specs=[pl.BlockSpec((BE, H), row), pl.BlockSpec((1, H), full),
                  pl.BlockSpec((H, 2), full), pl.BlockSpec((1, 2), full)],
        out_specs=pl.BlockSpec((BE, 2), row),
        out_shape=jax.ShapeDtypeStruct((E_PAD, 2), jnp.float32),
    )(zsum, b3, w4, b4)


def kernel(x_user, x_merchant, edge_index, Wl1, Wr1, b1, c1,
           Wl2, Wr2, b2, c2, W3, b3, W4, b4):
    src = edge_index[0]
    dst = edge_index[1]
    pad = E_PAD - E
    zpad = jnp.zeros((pad,), jnp.int32)
    spad = jnp.full((pad,), SENT, jnp.int32)
    src_g = jnp.concatenate([src, zpad])
    dst_g = jnp.concatenate([dst, zpad])
    src_gd = jnp.concatenate([src_g * 2, src_g * 2 + 1])
    dst_gd = jnp.concatenate([dst_g * 2, dst_g * 2 + 1])
    src_s2 = jnp.concatenate([src, spad])
    dst_s2 = jnp.concatenate([dst, spad])
    deg_idx = jnp.concatenate([dst_s2, src_s2])

    wl1e = jnp.einsum("ti,idh->tdh", c1, Wl1)
    wr1e = jnp.einsum("ti,idh->tdh", c1, Wr1)
    b1e = c1 @ b1
    wl2e = jnp.einsum("ti,idh->tdh", c2, Wl2)
    wr2e = jnp.einsum("ti,idh->tdh", c2, Wr2)
    b2e = c2 @ b2

    cnt = _degrees(deg_idx)

    npad = jnp.zeros((ACC - N, D), jnp.float32)
    pm, pu, rm, ru = _dense1(
        jnp.concatenate([x_user, npad]), jnp.concatenate([x_merchant, npad]),
        wl1e[0], wl1e[1], wr1e[0], wr1e[1], b1e)

    iv = lambda t: t.reshape(2 * ACC, 32)
    sm, su = _segsum(iv(pm), iv(pu), src_gd, src_s2, dst_gd, dst_s2)

    pm2, pu2, rm2, ru2 = _dense2(sm, su, cnt, rm, ru,
                                 wl2e[0], wl2e[1], wr2e[0], wr2e[1], b2e)

    sm2, su2 = _segsum(iv(pm2), iv(pu2), src_gd, src_s2, dst_gd, dst_s2)

    au, am = _dense3(sm2, su2, cnt, rm2, ru2, W3[:H], W3[H:])

    zsum = _edge_gather(au, am, src_g, dst_g)

    outp = _final(zsum, b3.reshape(1, H), W4, b4.reshape(1, 2))
    return outp[:E]
